# baseline (speedup 1.0000x reference)
import functools

import jax
import jax.numpy as jnp
import numpy as np
from jax import lax
from jax.experimental import pallas as pl
from jax.experimental.pallas import tpu as pltpu
from jax.experimental.pallas import tpu_sc as plsc

_ROT0 = (13, 15, 26, 6)
_ROT1 = (17, 29, 16, 24)


def _tf2x32(k1, k2, c1, c2):
    k1 = np.asarray(k1, np.uint32)
    k2 = np.asarray(k2, np.uint32)
    x0 = np.asarray(c1, np.uint32)
    x1 = np.asarray(c2, np.uint32)
    ks2 = k1 ^ k2 ^ np.uint32(0x1BD11BDA)

    def rnds(x0, x1, rots):
        for r in rots:
            x0 = (x0 + x1).astype(np.uint32)
            x1 = ((x1 << np.uint32(r)) | (x1 >> np.uint32(32 - r))).astype(np.uint32)
            x1 = x0 ^ x1
        return x0, x1

    x0 = (x0 + k1).astype(np.uint32)
    x1 = (x1 + k2).astype(np.uint32)
    x0, x1 = rnds(x0, x1, _ROT0)
    x0 = (x0 + k2).astype(np.uint32)
    x1 = (x1 + ks2 + np.uint32(1)).astype(np.uint32)
    x0, x1 = rnds(x0, x1, _ROT1)
    x0 = (x0 + ks2).astype(np.uint32)
    x1 = (x1 + k1 + np.uint32(2)).astype(np.uint32)
    x0, x1 = rnds(x0, x1, _ROT0)
    x0 = (x0 + k1).astype(np.uint32)
    x1 = (x1 + k2 + np.uint32(3)).astype(np.uint32)
    x0, x1 = rnds(x0, x1, _ROT1)
    x0 = (x0 + k2).astype(np.uint32)
    x1 = (x1 + ks2 + np.uint32(4)).astype(np.uint32)
    x0, x1 = rnds(x0, x1, _ROT0)
    x0 = (x0 + ks2).astype(np.uint32)
    x1 = (x1 + k1 + np.uint32(5)).astype(np.uint32)
    return x0, x1


def _np_fold_in(key, d):
    a, b = _tf2x32(key[0], key[1], np.zeros(1, np.uint32),
                   np.full(1, d, np.uint32))
    return a[0], b[0]


def _np_random_bits(key, n):
    b1, b2 = _tf2x32(key[0], key[1], np.zeros(n, np.uint32),
                     np.arange(n, dtype=np.uint32))
    return b1 ^ b2


def _np_uniform(key, n):
    bits = _np_random_bits(key, n)
    fb = ((bits >> np.uint32(9)) | np.uint32(0x3F800000)).astype(np.uint32)
    return fb.view(np.float32) - np.float32(1.0)


def _np_permutation(key, n):
    x = np.arange(n)
    for _ in range(2):
        b1, b2 = _tf2x32(key[0], key[1], np.zeros(2, np.uint32),
                         np.arange(2, dtype=np.uint32))
        key, sub = (b1[0], b2[0]), (b1[1], b2[1])
        x = x[np.argsort(_np_random_bits(sub, n), kind="stable")]
    return x


B = 16384
D = 512
SUBW = 128
NSUB = 4
LW = 1000
NB = 5
NW = 32
KE = 64
ECH = B // (NW * KE)
KL = 4
LCH = B // (NW * KL)
RPT = B // NW
OW = NB * LW


def _mesh():
    return plsc.VectorSubcoreMesh(core_axis_name="c", subcore_axis_name="s")


@functools.lru_cache(maxsize=None)
def _routing():
    key = (np.uint32(0), np.uint32(42))
    dec = _np_uniform(_np_fold_in(key, 0), B) < 0.5
    ps = [_np_permutation(_np_fold_in(key, i), B) for i in range(1, NSUB)]
    r = np.arange(B)

    esrc = (np.stack([r] + [np.where(dec, p, r) for p in ps])
            .astype(np.int32).reshape(NSUB, NW, ECH, KE)
            .transpose(1, 0, 2, 3).copy())

    lsrc = (np.stack([r, r] + [np.where(dec, p, r) for p in ps])
            .astype(np.int32).reshape(NB, NW, LCH, KL)
            .transpose(1, 0, 2, 3).copy())

    scl = np.empty((NW, RPT, 2, 16), np.float32)
    decf = dec.astype(np.float32).reshape(NW, RPT)
    scl[:, :, 0, :] = (1.0 - decf)[:, :, None]
    scl[:, :, 1, :] = (0.25 * decf)[:, :, None]

    return esrc, lsrc, scl


@functools.lru_cache(maxsize=None)
def _build_embed():
    def body(embed, esrc, out_e, esrc_v, eb0, eb1, gsem, wsem):
        wid = lax.axis_index("s") * 2 + lax.axis_index("c")
        base = wid * RPT
        pltpu.sync_copy(esrc.at[wid], esrc_v)

        def gather(c, eb):
            return [pltpu.async_copy(
                embed.at[esrc_v.at[i, c], pl.ds(i * SUBW, SUBW)],
                eb.at[:, pl.ds(i * SUBW, SUBW)], gsem)
                for i in range(NSUB)]

        def pair(p, carry):
            c0 = 2 * p
            g0 = gather(c0, eb0)
            for d in g0:
                d.wait()
            w0 = pltpu.async_copy(eb0, out_e.at[pl.ds(base + c0 * KE, KE)], wsem)
            g1 = gather(c0 + 1, eb1)
            for d in g1:
                d.wait()
            w0.wait()
            w1 = pltpu.async_copy(eb1, out_e.at[pl.ds(base + (c0 + 1) * KE, KE)], wsem)
            w1.wait()
            return carry
        lax.fori_loop(0, ECH // 2, pair, 0)

    return pl.kernel(
        body,
        out_type=jax.ShapeDtypeStruct((B, D), jnp.float32),
        mesh=_mesh(),
        scratch_types=[
            pltpu.VMEM((NSUB, ECH, KE), jnp.int32),
            pltpu.VMEM((KE, D), jnp.float32),
            pltpu.VMEM((KE, D), jnp.float32),
            pltpu.SemaphoreType.DMA,
            pltpu.SemaphoreType.DMA,
        ],
    )


@functools.lru_cache(maxsize=None)
def _build_label():
    def body(label, lsrc, scl, out_l,
             lsrc_v, scl_v, sa, sb, ca, cb, gsem, wsem):
        wid = lax.axis_index("s") * 2 + lax.axis_index("c")
        base = wid * RPT

        pltpu.sync_copy(lsrc.at[wid], lsrc_v)
        pltpu.sync_copy(scl.at[wid], scl_v)

        def gather(c, st):
            return [pltpu.async_copy(
                label.at[lsrc_v.at[j, c]],
                st.at[pl.ds(j * KL, KL)], gsem)
                for j in range(NB)]

        def movescale(c, st, co):
            def srow(i, carry):
                s0 = scl_v[c * KL + i, 0]
                s1 = scl_v[c * KL + i, 1]
                for j in range(NB):
                    s = s0 if j == 0 else s1
                    for t in range(62):
                        co[i, pl.ds(j * LW + t * 16, 16)] = \
                            st[j * KL + i, pl.ds(t * 16, 16)] * s
                    co[i, pl.ds(j * LW + LW - 16, 16)] = \
                        st[j * KL + i, pl.ds(LW - 16, 16)] * s
                return carry
            lax.fori_loop(0, KL, srow, 0)

        def pair(p, carry):
            c0 = 2 * p
            g0 = gather(c0, sa)
            g1 = gather(c0 + 1, sb)
            for d in g0:
                d.wait()
            movescale(c0, sa, ca)
            w0 = pltpu.async_copy(ca, out_l.at[pl.ds(base + c0 * KL, KL)], wsem)
            for d in g1:
                d.wait()
            movescale(c0 + 1, sb, cb)
            w0.wait()
            pltpu.async_copy(cb, out_l.at[pl.ds(base + (c0 + 1) * KL, KL)],
                             wsem).wait()
            return carry
        lax.fori_loop(0, LCH // 2, pair, 0)

    return pl.kernel(
        body,
        compiler_params=pltpu.CompilerParams(use_tc_tiling_on_sc=False),
        out_type=jax.ShapeDtypeStruct((B, OW), jnp.float32),
        mesh=_mesh(),
        scratch_types=[
            pltpu.VMEM((NB, LCH, KL), jnp.int32),
            pltpu.VMEM((RPT, 2, 16), jnp.float32),
            pltpu.VMEM((NB * KL, LW), jnp.float32),
            pltpu.VMEM((NB * KL, LW), jnp.float32),
            pltpu.VMEM((KL, OW), jnp.float32),
            pltpu.VMEM((KL, OW), jnp.float32),
            pltpu.SemaphoreType.DMA,
            pltpu.SemaphoreType.DMA,
        ],
    )


def kernel(embed, onehot_label):
    esrc, lsrc, scl = _routing()
    out_e = _build_embed()(embed, esrc)
    out_l = _build_label()(onehot_label, lsrc, scl)
    return out_e, out_l
